# baseline (speedup 1.0000x reference)
import jax
import jax.numpy as jnp
from jax.experimental import pallas as pl
from jax.experimental.pallas import tpu as pltpu

NUM_EXPERT = 64
IN_FEAT = 768
OUT_FEAT = 768
N_TOKENS = 128
EPG = 4
NSTEPS = NUM_EXPERT // EPG
NBUF = 3


def _moe_kernel(gate_ref, inp_ref, w_hbm, out_ref, scratch, sems):
    def dma(c, slot):
        return pltpu.make_async_copy(
            w_hbm.at[pl.ds(c * EPG, EPG)], scratch.at[slot], sems.at[slot]
        )

    for b in range(NBUF):
        dma(b, b).start()

    out_ref[...] = jnp.zeros_like(out_ref)
    inp = inp_ref[...]
    gate = gate_ref[...]

    def body(c, _):
        slot = jax.lax.rem(c, NBUF)
        dma(c, slot).wait()
        acc = out_ref[...]
        for j in range(EPG):
            e = c * EPG + j
            mask = gate == e
            x = jnp.where(mask, inp, 0.0)
            acc += jax.lax.dot_general(
                x, scratch[slot, j],
                (((1,), (1,)), ((), ())),
                preferred_element_type=jnp.float32,
            )
        out_ref[...] = acc

        @pl.when(c + NBUF < NSTEPS)
        def _next():
            dma(c + NBUF, slot).start()

        return 0

    jax.lax.fori_loop(0, NSTEPS, body, 0)


def kernel(inp, gate, weight):
    gate2d = gate.reshape(N_TOKENS, 1)
    return pl.pallas_call(
        _moe_kernel,
        in_specs=[
            pl.BlockSpec(memory_space=pltpu.MemorySpace.VMEM),
            pl.BlockSpec(memory_space=pltpu.MemorySpace.VMEM),
            pl.BlockSpec(memory_space=pltpu.MemorySpace.HBM),
        ],
        out_specs=pl.BlockSpec(memory_space=pltpu.MemorySpace.VMEM),
        out_shape=jax.ShapeDtypeStruct((N_TOKENS, OUT_FEAT), jnp.float32),
        scratch_shapes=[
            pltpu.VMEM((NBUF, EPG, OUT_FEAT, IN_FEAT), jnp.float32),
            pltpu.SemaphoreType.DMA((NBUF,)),
        ],
    )(gate2d, inp, weight)
